# Initial kernel scaffold; baseline (speedup 1.0000x reference)
import functools

import jax
import jax.numpy as jnp
from jax import lax
from jax.experimental import pallas as pl
from jax.experimental.pallas import tpu as pltpu
from jax.experimental.pallas import tpu_sc as plsc

NUM_CORES = 2
NUM_SUBCORES = 16
NUM_WORKERS = NUM_CORES * NUM_SUBCORES
CHUNK = 1024


def _emb_body(idx_hbm, table_hbm, out_hbm, idx_v, rows_v, sem):
    wid = lax.axis_index("s") * NUM_CORES + lax.axis_index("c")
    per_w = idx_hbm.shape[0] // NUM_WORKERS
    base = wid * per_w
    nchunks = per_w // CHUNK

    def body(g, carry):
        off = base + g * CHUNK
        pltpu.sync_copy(idx_hbm.at[pl.ds(off, CHUNK)], idx_v)
        pltpu.async_copy(table_hbm.at[idx_v], rows_v, sem).wait()
        pltpu.sync_copy(rows_v, out_hbm.at[pl.ds(off, CHUNK)])
        return carry

    lax.fori_loop(0, nchunks, body, 0)


@jax.jit
def _embedding_lookup(flat_ids, weight):
    b = flat_ids.shape[0]
    d = weight.shape[1]
    mesh = plsc.VectorSubcoreMesh(core_axis_name="c", subcore_axis_name="s")
    return pl.kernel(
        _emb_body,
        out_type=jax.ShapeDtypeStruct((b, d), weight.dtype),
        mesh=mesh,
        scratch_types=[
            pltpu.VMEM((CHUNK,), jnp.int32),
            pltpu.VMEM((CHUNK, d), jnp.float32),
            pltpu.SemaphoreType.DMA,
        ],
    )(flat_ids, weight)


def kernel(token_ids, weight):
    flat = token_ids.reshape(-1).astype(jnp.int32)
    out = _embedding_lookup(flat, weight)
    return out.reshape(*token_ids.shape, weight.shape[1])


# speedup vs baseline: 1.4589x; 1.4589x over previous
import functools

import jax
import jax.numpy as jnp
from jax import lax
from jax.experimental import pallas as pl
from jax.experimental.pallas import tpu as pltpu
from jax.experimental.pallas import tpu_sc as plsc

NUM_CORES = 2
NUM_SUBCORES = 16
NUM_WORKERS = NUM_CORES * NUM_SUBCORES
CHUNK = 1024


def _emb_body(idx_hbm, table_hbm, out_hbm, idx_v, rows_v, sem):
    wid = lax.axis_index("s") * NUM_CORES + lax.axis_index("c")
    per_w = idx_hbm.shape[0] // NUM_WORKERS
    base = wid * per_w
    nchunks = per_w // CHUNK

    def body(g, carry):
        off = base + g * CHUNK
        pltpu.sync_copy(idx_hbm.at[pl.ds(off, CHUNK)], idx_v)
        pltpu.async_copy(table_hbm.at[idx_v], rows_v, sem).wait()
        pltpu.sync_copy(rows_v, out_hbm.at[pl.ds(off, CHUNK)])
        return carry

    lax.fori_loop(0, nchunks, body, 0)


@jax.jit
def _embedding_lookup(flat_ids, weight):
    b = flat_ids.shape[0]
    d = weight.shape[1]
    mesh = plsc.VectorSubcoreMesh(core_axis_name="c", subcore_axis_name="s")
    return pl.kernel(
        _emb_body,
        out_type=jax.ShapeDtypeStruct((b, d), weight.dtype),
        mesh=mesh,
        scratch_types=[
            pltpu.VMEM((CHUNK,), jnp.int32),
            pltpu.VMEM((CHUNK, d), jnp.float32),
            pltpu.SemaphoreType.DMA,
        ],
        compiler_params=pltpu.CompilerParams(use_tc_tiling_on_sc=False),
    )(flat_ids, weight)


def kernel(token_ids, weight):
    flat = token_ids.reshape(-1).astype(jnp.int32)
    out = _embedding_lookup(flat, weight)
    return out.reshape(*token_ids.shape, weight.shape[1])


# speedup vs baseline: 1.5027x; 1.0301x over previous
import functools

import jax
import jax.numpy as jnp
from jax import lax
from jax.experimental import pallas as pl
from jax.experimental.pallas import tpu as pltpu
from jax.experimental.pallas import tpu_sc as plsc

NUM_CORES = 2
NUM_SUBCORES = 16
NUM_WORKERS = NUM_CORES * NUM_SUBCORES
CHUNK = 1280
N_BUF = 2


def _emb_body(idx_hbm, table_hbm, out_hbm, *scratch):
    idx_v = scratch[0]
    rows = scratch[1 : 1 + N_BUF]
    gsem = scratch[1 + N_BUF : 1 + 2 * N_BUF]
    ssem = scratch[1 + 2 * N_BUF : 1 + 3 * N_BUF]

    wid = lax.axis_index("s") * NUM_CORES + lax.axis_index("c")
    n = idx_hbm.shape[0] // NUM_WORKERS
    base = wid * n * CHUNK
    idx2 = idx_v

    pltpu.sync_copy(idx_hbm.at[pl.ds(wid * n, n)], idx_v)

    def gather_start(g, b):
        pltpu.make_async_copy(table_hbm.at[idx2.at[g]], rows[b], gsem[b]).start()

    def store_start(g, b):
        pltpu.make_async_copy(
            rows[b], out_hbm.at[pl.ds(base + g * CHUNK, CHUNK)], ssem[b]
        ).start()

    def gather_wait(b):
        pltpu.make_async_copy(table_hbm.at[idx2.at[0]], rows[b], gsem[b]).wait()

    def store_wait(g, b):
        pltpu.make_async_copy(
            rows[b], out_hbm.at[pl.ds(base + g * CHUNK, CHUNK)], ssem[b]
        ).wait()

    for b in range(N_BUF):
        gather_start(b, b)

    def body(p, carry):
        g0 = p * N_BUF
        for b in range(N_BUF):
            g = g0 + b
            gather_wait(b)
            store_start(g, b)
            store_wait(g, b)
            gather_start(g + N_BUF, b)
        return carry

    lax.fori_loop(0, n // N_BUF - 1, body, 0)

    for b in range(N_BUF):
        g = n - N_BUF + b
        gather_wait(b)
        store_start(g, b)
        store_wait(g, b)


@jax.jit
def _embedding_lookup(flat_ids, weight):
    b = flat_ids.shape[0]
    d = weight.shape[1]
    per_w = b // NUM_WORKERS
    n = per_w // CHUNK
    ids2 = flat_ids.reshape(b // CHUNK, CHUNK)
    mesh = plsc.VectorSubcoreMesh(core_axis_name="c", subcore_axis_name="s")
    scratch = [pltpu.VMEM((n, CHUNK), jnp.int32)]
    scratch += [pltpu.VMEM((CHUNK, d), jnp.float32) for _ in range(N_BUF)]
    scratch += [pltpu.SemaphoreType.DMA for _ in range(2 * N_BUF)]
    return pl.kernel(
        _emb_body,
        out_type=jax.ShapeDtypeStruct((b, d), weight.dtype),
        mesh=mesh,
        scratch_types=scratch,
        compiler_params=pltpu.CompilerParams(use_tc_tiling_on_sc=False),
    )(ids2, weight)


def kernel(token_ids, weight):
    flat = token_ids.reshape(-1).astype(jnp.int32)
    out = _embedding_lookup(flat, weight)
    return out.reshape(*token_ids.shape, weight.shape[1])
